# Initial kernel scaffold; baseline (speedup 1.0000x reference)
import jax
import jax.numpy as jnp
from jax.experimental import pallas as pl


def kernel(x, positional_emb):
    raise NotImplementedError("write your pallas kernel here")


# speedup vs baseline: 3.0497x; 3.0497x over previous
import functools

import jax
import jax.numpy as jnp
from jax import lax
from jax.experimental import pallas as pl
from jax.experimental.pallas import tpu as pltpu
from jax.experimental.pallas import tpu_sc as plsc


@functools.lru_cache(maxsize=None)
def _make_sc_bcast(b, t, d):
    info = plsc.get_sparse_core_info()
    nc, ns = info.num_cores, info.num_subcores
    nw = nc * ns
    assert t % nw == 0
    rows_per_w = t // nw
    ch = rows_per_w
    while ch * d * 4 > 256 * 1024:
        ch //= 2
    n_ch = rows_per_w // ch

    mesh = plsc.VectorSubcoreMesh(core_axis_name="c", subcore_axis_name="s")

    @functools.partial(
        pl.kernel,
        mesh=mesh,
        out_type=jax.ShapeDtypeStruct((b, t, d), jnp.float32),
        scratch_types=[
            pltpu.VMEM((ch, d), jnp.float32),
            pltpu.SemaphoreType.DMA,
        ],
    )
    def k(table_hbm, out_hbm, buf, sem):
        wid = lax.axis_index("s") * nc + lax.axis_index("c")
        base = wid * rows_per_w
        for i in range(n_ch):
            r0 = base + i * ch
            pltpu.sync_copy(table_hbm.at[pl.ds(r0, ch)], buf)
            copies = [
                pltpu.async_copy(buf, out_hbm.at[bb, pl.ds(r0, ch)], sem)
                for bb in range(b)
            ]
            for c in copies:
                c.wait()

    return k


def kernel(x, positional_emb):
    b, t = x.shape
    d = positional_emb.shape[1]
    return _make_sc_bcast(b, t, d)(positional_emb)
